# Initial kernel scaffold; baseline (speedup 1.0000x reference)
import jax
import jax.numpy as jnp
from jax.experimental import pallas as pl


def kernel(x):
    raise NotImplementedError("write your pallas kernel here")


# speedup vs baseline: 197.2865x; 197.2865x over previous
import jax
import jax.numpy as jnp
from jax.experimental import pallas as pl


def _stencil_kernel(x_ref, o_ref):
    xb = x_ref[...]
    B, D, H, W = xb.shape
    R = B * D * H
    x2 = xb.reshape(R, W)
    z1 = jnp.zeros((R, 1), dtype=x2.dtype)
    z2 = jnp.zeros((R, 2), dtype=x2.dtype)
    l1 = jnp.concatenate([z1, x2[:, : W - 1]], axis=1)
    l2 = jnp.concatenate([z2, x2[:, : W - 2]], axis=1)
    r1 = jnp.concatenate([x2[:, 1:], z1], axis=1)
    r2 = jnp.concatenate([x2[:, 2:], z2], axis=1)
    w_idx = jax.lax.broadcasted_iota(jnp.int32, (R, W), 1)
    even = (w_idx % 2) == 0
    one = jnp.ones((), dtype=x2.dtype)
    zero = jnp.zeros((), dtype=x2.dtype)
    sel_e = jnp.where((x2 > l1) & (x2 >= r1), one, zero)
    sel_o = jnp.where((x2 >= r1) & (x2 >= r2), one, zero)
    sel_o = jnp.maximum(sel_o, jnp.where((x2 > l2) & (x2 > l1), one, zero))
    sel = jnp.where(even, sel_e, sel_o)
    soft = x2 / (1.0 + jnp.abs(x2))
    contrib = x2 + soft * sel
    core = jnp.sum(contrib.reshape(B, D, H, W), axis=1) * (1.0 / 17.0)
    zh = jnp.zeros((B, 1, W), dtype=core.dtype)
    full_h = jnp.concatenate([zh, core, zh], axis=1)
    zl = jnp.zeros((B, H + 2, 1), dtype=core.dtype)
    zr = jnp.zeros((B, H + 2, 2), dtype=core.dtype)
    o_ref[...] = jnp.concatenate([zl, full_h, zr], axis=2)


def kernel(x):
    N, C, D, H, W = x.shape
    NC = N * C
    B = 4
    xr = x.reshape(NC, D, H, W)
    out = pl.pallas_call(
        _stencil_kernel,
        grid=(NC // B,),
        in_specs=[pl.BlockSpec((B, D, H, W), lambda i: (i, 0, 0, 0))],
        out_specs=pl.BlockSpec((B, H + 2, W + 3), lambda i: (i, 0, 0)),
        out_shape=jax.ShapeDtypeStruct((NC, H + 2, W + 3), x.dtype),
    )(xr)
    return out.reshape(N, C, H + 2, W + 3)
